# baseline (speedup 1.0000x reference)
import functools

import jax
import jax.numpy as jnp
from jax import lax
from jax.experimental import pallas as pl
from jax.experimental.pallas import tpu as pltpu
from jax.experimental.pallas import tpu_sc as plsc

_L = 16
_CHUNK = 128


def kernel(heads, rels, entity_table, rel_table):
    B = heads.shape[0]
    E, D = entity_table.shape
    R = rel_table.shape[0]
    PACK = 128 // D

    info = plsc.get_sparse_core_info()
    NC, NS = info.num_cores, info.num_subcores
    NW = NC * NS
    b_w = B // NW
    n_chunks = b_w // _CHUNK
    n_groups = b_w // _L
    assert b_w * NW == B and n_chunks * _CHUNK == b_w

    etab = entity_table.reshape(E // PACK, 128)
    rtab = rel_table.reshape(R // PACK, 128)

    mesh = plsc.VectorSubcoreMesh(core_axis_name="c", subcore_axis_name="s")

    @functools.partial(
        pl.kernel,
        mesh=mesh,
        compiler_params=pltpu.CompilerParams(needs_layout_passes=False),
        out_type=(
            jax.ShapeDtypeStruct((B * D,), jnp.float32),
            jax.ShapeDtypeStruct((B * D,), jnp.float32),
        ),
        scratch_types=[
            pltpu.VMEM((b_w,), jnp.int32),
            pltpu.VMEM((b_w,), jnp.int32),
            pltpu.VMEM((b_w,), jnp.int32),
            pltpu.VMEM((R // PACK, 128), jnp.float32),
            pltpu.VMEM((2 * _CHUNK, 128), jnp.float32),
            pltpu.VMEM((b_w * D,), jnp.float32),
            pltpu.VMEM((b_w * D,), jnp.float32),
            pltpu.SemaphoreType.DMA,
            pltpu.SemaphoreType.DMA,
            pltpu.SemaphoreType.DMA,
        ],
    )
    def _gather2(heads_hbm, rels_hbm, etab_hbm, rtab_hbm, out_h_hbm, out_r_hbm,
                 hidx, hdiv, ridx, reltab, hrows, hout, rout,
                 sem_i, sem_r, sem_g):
        wid = lax.axis_index("s") * NC + lax.axis_index("c")
        base = wid * b_w
        iota = lax.iota(jnp.int32, _L)

        idx_copies = []
        for j in range(n_chunks):
            idx_copies.append(pltpu.async_copy(
                heads_hbm.at[pl.ds(base + j * _CHUNK, _CHUNK)],
                hidx.at[pl.ds(j * _CHUNK, _CHUNK)], sem_i))
            idx_copies.append(pltpu.async_copy(
                rels_hbm.at[pl.ds(base + j * _CHUNK, _CHUNK)],
                ridx.at[pl.ds(j * _CHUNK, _CHUNK)], sem_i))
        rel_copy = pltpu.async_copy(rtab_hbm, reltab, sem_r)
        for c in idx_copies:
            c.wait()

        def _div_body(i, _):
            v = hidx[pl.ds(i * _L, _L)]
            hdiv[pl.ds(i * _L, _L)] = v >> 2
            return 0
        lax.fori_loop(0, b_w // _L, _div_body, 0, unroll=4)

        def _extract_entity(g, half):
            grow = half * 2 * _CHUNK + g * _L
            idxv = hidx[pl.ds(grow, _L)]
            colb = (idxv & (PACK - 1)) * D
            rowv = g * _L + iota
            fbase = grow * D + iota * D
            for c in range(D):
                v = plsc.load_gather(hrows, [rowv, colb + c])
                plsc.store_scatter(hout, [fbase + c], v)
            return half

        def _extract_rel(g, _):
            grow = g * _L
            relv = ridx[pl.ds(grow, _L)]
            rrow = relv >> 2
            colb = (relv & (PACK - 1)) * D
            fbase = grow * D + iota * D
            for c in range(D):
                v = plsc.load_gather(reltab, [rrow, colb + c])
                plsc.store_scatter(rout, [fbase + c], v)
            return 0

        for half in range(2):
            g0 = pltpu.async_copy(
                etab_hbm.at[hdiv.at[pl.ds((2 * half) * _CHUNK, _CHUNK)]],
                hrows.at[pl.ds(0, _CHUNK)], sem_g)
            g1 = pltpu.async_copy(
                etab_hbm.at[hdiv.at[pl.ds((2 * half + 1) * _CHUNK, _CHUNK)]],
                hrows.at[pl.ds(_CHUNK, _CHUNK)], sem_g)
            g0.wait()
            g1.wait()
            lax.fori_loop(0, (2 * _CHUNK) // _L,
                          lambda g, c: _extract_entity(g, c), half)

        rel_copy.wait()
        lax.fori_loop(0, n_groups, _extract_rel, 0)

        pltpu.sync_copy(hout, out_h_hbm.at[pl.ds(base * D, b_w * D)])
        pltpu.sync_copy(rout, out_r_hbm.at[pl.ds(base * D, b_w * D)])

    out_h, out_r = _gather2(heads, rels, etab, rtab)
    return (out_h.reshape(B, D), out_r.reshape(B, D))


# speedup vs baseline: 3.9145x; 3.9145x over previous
import functools

import jax
import jax.numpy as jnp
from jax import lax
from jax.experimental import pallas as pl
from jax.experimental.pallas import tpu as pltpu
from jax.experimental.pallas import tpu_sc as plsc

_L = 16


def kernel(heads, rels, entity_table, rel_table):
    B = heads.shape[0]
    E, D = entity_table.shape
    R = rel_table.shape[0]
    NP, NC8 = D // 8, 8
    RT = R // 128
    RTAIL = R - RT * 128

    info = plsc.get_sparse_core_info()
    NW = info.num_cores * info.num_subcores
    b_w = B // NW
    n_grp = b_w // _L
    assert b_w * NW == B and n_grp * _L == b_w

    etT = entity_table.T.reshape(NP, NC8, E)
    rtT = rel_table.T

    mesh = plsc.VectorSubcoreMesh(core_axis_name="c", subcore_axis_name="s")

    @functools.partial(
        pl.kernel,
        mesh=mesh,
        compiler_params=pltpu.CompilerParams(needs_layout_passes=False),
        out_type=(
            jax.ShapeDtypeStruct((B * D,), jnp.float32),
            jax.ShapeDtypeStruct((B * D,), jnp.float32),
        ),
        scratch_types=[
            pltpu.VMEM((b_w,), jnp.int32),
            pltpu.VMEM((b_w,), jnp.int32),
            pltpu.VMEM((NP * (RT + 1), 8, 128), jnp.float32),
            pltpu.VMEM((2, _L * D * _L), jnp.float32),
            pltpu.VMEM((b_w * D,), jnp.float32),
            pltpu.VMEM((b_w * D,), jnp.float32),
            pltpu.SemaphoreType.DMA,
            pltpu.SemaphoreType.DMA,
            pltpu.SemaphoreType.DMA,
        ],
    )
    def _gather2(heads_hbm, rels_hbm, etT_hbm, rtT_hbm, out_h_hbm, out_r_hbm,
                 hidx_v, ridx_v, relv, gbuf, hout, rout, sem_i, sem_r, sem_g):
        wid = lax.axis_index("s") * info.num_cores + lax.axis_index("c")
        base = wid * b_w
        iota = lax.iota(jnp.int32, _L)
        a16 = iota * _L
        grp_words = _L * D * _L

        ci = pltpu.async_copy(heads_hbm.at[pl.ds(base, b_w)], hidx_v, sem_i)
        cr = pltpu.async_copy(rels_hbm.at[pl.ds(base, b_w)], ridx_v, sem_i)

        rel_copies = []
        for i in range(NP):
            for j in range(RT):
                rel_copies.append(pltpu.async_copy(
                    rtT_hbm.at[pl.ds(8 * i, 8), pl.ds(128 * j, 128)],
                    relv.at[i * (RT + 1) + j], sem_r))
        if RTAIL:
            for c in range(D):
                rel_copies.append(pltpu.async_copy(
                    rtT_hbm.at[c, pl.ds(RT * 128, RTAIL)],
                    relv.at[(c // 8) * (RT + 1) + RT, c % 8, pl.ds(0, RTAIL)],
                    sem_r))

        ci.wait()
        cr.wait()

        def _fire(g):
            slot = g & 1
            idxv = hidx_v[pl.ds(g * _L, _L)]
            for lane in range(_L):
                r = idxv[lane]
                gg = pl.multiple_of((r >> 4) * _L, _L)
                for k in range(D):
                    pltpu.async_copy(
                        etT_hbm.at[k // 8, k % 8, pl.ds(gg, _L)],
                        gbuf.at[slot, pl.ds(lane * (D * _L) + k * _L, _L)],
                        sem_g)

        def _drain_extract(g):
            slot = g & 1
            pltpu.make_async_copy(
                out_h_hbm.at[pl.ds(0, grp_words)], gbuf.at[slot], sem_g).wait()
            idxv = hidx_v[pl.ds(g * _L, _L)]
            rmv = idxv & 15
            slotv = iota * 0 + slot
            for lane in range(_L):
                addr0 = a16 + (lane * (D * _L) + rmv[lane])
                v0 = plsc.load_gather(gbuf, [slotv, addr0])
                v1 = plsc.load_gather(gbuf, [slotv, addr0 + _L * _L])
                hout[pl.ds(g * (_L * D) + lane * D, _L)] = v0
                hout[pl.ds(g * (_L * D) + lane * D + _L, _L)] = v1

        _fire(jnp.int32(0))

        def _steady(g, _):
            pltpu.make_async_copy(
                out_h_hbm.at[pl.ds(0, grp_words)],
                gbuf.at[g & 1], sem_g).wait()
            _fire(g + 1)
            _drain_extract_body(g)
            return 0

        def _drain_extract_body(g):
            slot = g & 1
            idxv = hidx_v[pl.ds(g * _L, _L)]
            rmv = idxv & 15
            slotv = iota * 0 + slot
            for lane in range(_L):
                addr0 = a16 + (lane * (D * _L) + rmv[lane])
                v0 = plsc.load_gather(gbuf, [slotv, addr0])
                v1 = plsc.load_gather(gbuf, [slotv, addr0 + _L * _L])
                hout[pl.ds(g * (_L * D) + lane * D, _L)] = v0
                hout[pl.ds(g * (_L * D) + lane * D + _L, _L)] = v1

        lax.fori_loop(0, n_grp - 1, _steady, 0)
        _drain_extract(jnp.int32(n_grp - 1))

        for c in rel_copies:
            c.wait()

        def _rel_group(g, _):
            rrv = ridx_v[pl.ds(g * _L, _L)]
            t = rrv >> 7
            m = rrv & 127
            outb = (g * _L + iota) * D
            for c2 in range(D):
                i0 = t + (c2 // 8) * (RT + 1)
                c8v = iota * 0 + (c2 % 8)
                v = plsc.load_gather(relv, [i0, c8v, m])
                plsc.store_scatter(rout, [outb + c2], v)
            return 0
        lax.fori_loop(0, n_grp, _rel_group, 0)

        pltpu.sync_copy(hout, out_h_hbm.at[pl.ds(base * D, b_w * D)])
        pltpu.sync_copy(rout, out_r_hbm.at[pl.ds(base * D, b_w * D)])

    out_h, out_r = _gather2(heads, rels, etT, rtT)
    return (out_h.reshape(B, D), out_r.reshape(B, D))


# speedup vs baseline: 3.9443x; 1.0076x over previous
import functools

import jax
import jax.numpy as jnp
from jax import lax
from jax.experimental import pallas as pl
from jax.experimental.pallas import tpu as pltpu
from jax.experimental.pallas import tpu_sc as plsc

_L = 16
_NS = 4


def kernel(heads, rels, entity_table, rel_table):
    B = heads.shape[0]
    E, D = entity_table.shape
    R = rel_table.shape[0]
    NP, NC8 = D // 8, 8
    RT = R // 128
    RTAIL = R - RT * 128

    info = plsc.get_sparse_core_info()
    NW = info.num_cores * info.num_subcores
    b_w = B // NW
    n_grp = b_w // _L
    tpw = b_w // 128
    assert b_w * NW == B and n_grp * _L == b_w

    etT = entity_table.T.reshape(NP, NC8, E)
    rtT = rel_table.T

    mesh = plsc.VectorSubcoreMesh(core_axis_name="c", subcore_axis_name="s")

    @functools.partial(
        pl.kernel,
        mesh=mesh,
        compiler_params=pltpu.CompilerParams(needs_layout_passes=False),
        out_type=(
            jax.ShapeDtypeStruct((B * D,), jnp.float32),
            jax.ShapeDtypeStruct((B * D,), jnp.float32),
        ),
        scratch_types=[
            pltpu.VMEM((b_w,), jnp.int32),
            pltpu.VMEM((b_w,), jnp.int32),
            pltpu.VMEM((NP * (RT + 1), 8, 128), jnp.float32),
            pltpu.VMEM((_NS, _L * D * _L), jnp.float32),
            pltpu.VMEM((b_w * D,), jnp.float32),
            pltpu.VMEM((b_w * D,), jnp.float32),
            pltpu.SemaphoreType.DMA,
            pltpu.SemaphoreType.DMA,
            pltpu.SemaphoreType.DMA,
        ],
    )
    def _gather2(heads_hbm, rels_hbm, etT_hbm, rtT_hbm,
                 out_h_hbm, out_r_hbm,
                 hidx_v, ridx_v, relv, gbuf, houtT, routT,
                 sem_i, sem_r, sem_g):
        wid = lax.axis_index("s") * info.num_cores + lax.axis_index("c")
        base = wid * b_w
        iota = lax.iota(jnp.int32, _L)
        c8lo = iota & 7
        tp0 = (iota >> 3) * tpw
        tp1 = ((iota + _L) >> 3) * tpw

        ci = pltpu.async_copy(heads_hbm.at[pl.ds(base, b_w)], hidx_v, sem_i)
        cr = pltpu.async_copy(rels_hbm.at[pl.ds(base, b_w)], ridx_v, sem_i)

        rel_copies = []
        for i in range(NP):
            for j in range(RT):
                rel_copies.append(pltpu.async_copy(
                    rtT_hbm.at[pl.ds(8 * i, 8), pl.ds(128 * j, 128)],
                    relv.at[i * (RT + 1) + j], sem_r))
        if RTAIL:
            for c in range(D):
                rel_copies.append(pltpu.async_copy(
                    rtT_hbm.at[c, pl.ds(RT * 128, RTAIL)],
                    relv.at[(c // 8) * (RT + 1) + RT, c % 8, pl.ds(0, RTAIL)],
                    sem_r))

        ci.wait()
        cr.wait()
        for c in rel_copies:
            c.wait()

        def _fire(g):
            slot = g & (_NS - 1)
            idxv = hidx_v[pl.ds(g * _L, _L)]
            for lane in range(_L):
                r = idxv[lane]
                gg = pl.multiple_of((r >> 4) * _L, _L)
                for k in range(D):
                    pltpu.async_copy(
                        etT_hbm.at[k // 8, k % 8, pl.ds(gg, _L)],
                        gbuf.at[slot, pl.ds(lane * (D * _L) + k * _L, _L)],
                        sem_g)

        def _extract(g):
            slot = g & (_NS - 1)
            idxv = hidx_v[pl.ds(g * _L, _L)]
            rmv = idxv & 15
            slotv = iota * 0 + slot
            a16 = iota * _L
            for lane in range(_L):
                rms = iota * 0 + rmv[lane]
                addr0 = a16 + (lane * (D * _L)) + rms
                v0 = plsc.load_gather(gbuf, [slotv, addr0])
                v1 = plsc.load_gather(gbuf, [slotv, addr0 + _L * _L])
                houtT[pl.ds(g * (_L * D) + lane * D, _L)] = v0
                houtT[pl.ds(g * (_L * D) + lane * D + _L, _L)] = v1

        def _drain(g):
            pltpu.make_async_copy(
                out_h_hbm.at[pl.ds(0, _L * D * _L)],
                gbuf.at[g & (_NS - 1)], sem_g).wait()

        def _rel_group(g):
            rrv = ridx_v[pl.ds(g * _L, _L)]
            t = rrv >> 7
            m = rrv & 127
            outb = (g * _L + iota) * D
            for c2 in range(D):
                i0 = t + (c2 // 8) * (RT + 1)
                c8v = iota * 0 + (c2 % 8)
                v = plsc.load_gather(relv, [i0, c8v, m])
                plsc.store_scatter(routT, [outb + c2], v)

        for g in range(_NS - 1):
            _fire(jnp.int32(g))

        def _steady(g, _):
            _drain(g)
            _fire(g + (_NS - 1))
            _extract(g)
            _rel_group(g)
            return 0
        lax.fori_loop(0, n_grp - (_NS - 1), _steady, 0)

        def _epi(g, _):
            _drain(g)
            _extract(g)
            _rel_group(g)
            return 0
        lax.fori_loop(n_grp - (_NS - 1), n_grp, _epi, 0)

        pltpu.sync_copy(houtT, out_h_hbm.at[pl.ds(base * D, b_w * D)])
        pltpu.sync_copy(routT, out_r_hbm.at[pl.ds(base * D, b_w * D)])

    out_h, out_r = _gather2(heads, rels, etT, rtT)
    return (out_h.reshape(B, D), out_r.reshape(B, D))
